# Initial kernel scaffold; baseline (speedup 1.0000x reference)
import jax
import jax.numpy as jnp
from jax.experimental import pallas as pl


def kernel(tensor, table):
    raise NotImplementedError("write your pallas kernel here")


# speedup vs baseline: 4.6576x; 4.6576x over previous
import functools

import jax
import jax.numpy as jnp
from jax import lax
from jax.experimental import pallas as pl
from jax.experimental.pallas import tpu as pltpu
from jax.experimental.pallas import tpu_sc as plsc

DIM = 32
NC = 2
NS = 16
NW = NC * NS
SUB = 128
K = 8
CHUNK = SUB * K


def _build(b_total: int):
    assert b_total % (NW * CHUNK) == 0
    n_chunks = b_total // (NW * CHUNK)
    idx_rows = b_total // SUB

    mesh = plsc.VectorSubcoreMesh(core_axis_name="c", subcore_axis_name="s")

    @functools.partial(
        pl.kernel,
        mesh=mesh,
        compiler_params=pltpu.CompilerParams(use_tc_tiling_on_sc=False),
        out_type=jax.ShapeDtypeStruct((b_total, DIM), jnp.float32),
        scratch_types=[
            pltpu.VMEM((K, SUB), jnp.int32),
            pltpu.VMEM((CHUNK, DIM), jnp.float32),
            pltpu.SemaphoreType.DMA,
        ],
    )
    def emb(idx_hbm, table_hbm, out_hbm, idx_v, rows_v, sem):
        wid = lax.axis_index("s") * NC + lax.axis_index("c")
        base = wid * (n_chunks * K)

        def body(i, carry):
            r0 = base + i * K
            pltpu.sync_copy(idx_hbm.at[pl.ds(r0, K)], idx_v)
            copies = [
                pltpu.async_copy(
                    table_hbm.at[idx_v.at[j]],
                    rows_v.at[pl.ds(j * SUB, SUB)],
                    sem,
                )
                for j in range(K)
            ]
            for c in copies:
                c.wait()
            pltpu.sync_copy(rows_v, out_hbm.at[pl.ds(r0 * SUB, CHUNK)])
            return carry

        lax.fori_loop(0, n_chunks, body, 0)

    return emb


def kernel(tensor, table):
    batch, hist = tensor.shape
    b_total = batch * hist
    idx2d = tensor.reshape(b_total // SUB, SUB)
    out = _build(b_total)(idx2d, table)
    return out.reshape(batch, hist, DIM)


# speedup vs baseline: 4.6627x; 1.0011x over previous
import functools

import jax
import jax.numpy as jnp
from jax import lax
from jax.experimental import pallas as pl
from jax.experimental.pallas import tpu as pltpu
from jax.experimental.pallas import tpu_sc as plsc

DIM = 32
NC = 2
NS = 16
NW = NC * NS
SUB = 128
K = 10
CHUNK = SUB * K


def _build(b_total: int):
    assert b_total % (NW * 2 * CHUNK) == 0
    n_chunks = b_total // (NW * CHUNK)
    n_pairs = n_chunks // 2
    idx_rows_pw = b_total // (NW * SUB)

    mesh = plsc.VectorSubcoreMesh(core_axis_name="c", subcore_axis_name="s")

    @functools.partial(
        pl.kernel,
        mesh=mesh,
        compiler_params=pltpu.CompilerParams(use_tc_tiling_on_sc=False),
        out_type=jax.ShapeDtypeStruct((b_total, DIM), jnp.float32),
        scratch_types=[
            pltpu.VMEM((idx_rows_pw, SUB), jnp.int32),
            pltpu.VMEM((CHUNK, DIM), jnp.float32),
            pltpu.VMEM((CHUNK, DIM), jnp.float32),
            pltpu.SemaphoreType.DMA,
            pltpu.SemaphoreType.DMA,
            pltpu.SemaphoreType.DMA,
            pltpu.SemaphoreType.DMA,
        ],
    )
    def emb(idx_hbm, table_hbm, out_hbm, idx_v, rows0, rows1,
            gat0, gat1, wb0, wb1):
        wid = lax.axis_index("s") * NC + lax.axis_index("c")
        ibase = wid * idx_rows_pw
        obase = ibase * SUB
        pltpu.sync_copy(idx_hbm.at[pl.ds(ibase, idx_rows_pw)], idx_v)

        def start_gathers(chunk, rows, sem):
            for j in range(K):
                pltpu.async_copy(
                    table_hbm.at[idx_v.at[chunk * K + j]],
                    rows.at[pl.ds(j * SUB, SUB)],
                    sem,
                )

        def wait_gathers(rows, sem):
            pltpu.make_async_copy(
                out_hbm.at[pl.ds(0, CHUNK)], rows, sem).wait()

        def start_wb(chunk, rows, sem):
            pltpu.async_copy(
                rows, out_hbm.at[pl.ds(obase + chunk * CHUNK, CHUNK)], sem)

        def wait_wb(rows, sem):
            pltpu.make_async_copy(
                rows, out_hbm.at[pl.ds(0, CHUNK)], sem).wait()

        start_gathers(0, rows0, gat0)

        def body(g, carry):
            e = 2 * g
            wait_gathers(rows0, gat0)

            @pl.when(g > 0)
            def _():
                wait_wb(rows1, wb1)

            start_gathers(e + 1, rows1, gat1)
            start_wb(e, rows0, wb0)
            wait_gathers(rows1, gat1)
            wait_wb(rows0, wb0)

            @pl.when(g < n_pairs - 1)
            def _():
                start_gathers(e + 2, rows0, gat0)

            start_wb(e + 1, rows1, wb1)
            return carry

        lax.fori_loop(0, n_pairs, body, 0)
        wait_wb(rows1, wb1)

    return emb


def kernel(tensor, table):
    batch, hist = tensor.shape
    b_total = batch * hist
    idx2d = tensor.reshape(b_total // SUB, SUB)
    out = _build(b_total)(idx2d, table)
    return out.reshape(batch, hist, DIM)


# speedup vs baseline: 5.0268x; 1.0781x over previous
import functools

import jax
import jax.numpy as jnp
from jax import lax
from jax.experimental import pallas as pl
from jax.experimental.pallas import tpu as pltpu
from jax.experimental.pallas import tpu_sc as plsc

DIM = 32
NC = 2
NS = 16
NW = NC * NS
GRP = 16
CHUNK = 800


def _build(b_total: int):
    assert b_total % (NW * 2 * CHUNK) == 0 and CHUNK % GRP == 0
    rows_pw = b_total // NW
    n_chunks = rows_pw // CHUNK
    n_pairs = n_chunks // 2
    n_groups = CHUNK // GRP

    mesh = plsc.VectorSubcoreMesh(core_axis_name="c", subcore_axis_name="s")

    @functools.partial(
        pl.kernel,
        mesh=mesh,
        compiler_params=pltpu.CompilerParams(
            use_tc_tiling_on_sc=False, needs_layout_passes=False),
        out_type=jax.ShapeDtypeStruct((b_total * DIM,), jnp.float32),
        scratch_types=[
            pltpu.VMEM((1000 * DIM,), jnp.float32),
            pltpu.VMEM((rows_pw,), jnp.int32),
            pltpu.VMEM((CHUNK * DIM,), jnp.float32),
            pltpu.VMEM((CHUNK * DIM,), jnp.float32),
            pltpu.SemaphoreType.DMA,
            pltpu.SemaphoreType.DMA,
        ],
    )
    def emb(idx_hbm, table_hbm, out_hbm, table_v, idx_v, rows0, rows1,
            wb0, wb1):
        wid = lax.axis_index("s") * NC + lax.axis_index("c")
        rbase = wid * rows_pw
        pltpu.sync_copy(table_hbm, table_v)
        pltpu.sync_copy(idx_hbm.at[pl.ds(rbase, rows_pw)], idx_v)

        iota = lax.iota(jnp.int32, 16)

        def compute_chunk(chunk, rows):
            coff = chunk * CHUNK

            def grp_body(gi, carry):
                row0 = gi * GRP
                base = idx_v[pl.ds(coff + row0, GRP)] * DIM
                for r in range(GRP):
                    lo = jnp.full((16,), base[r], jnp.int32) + iota
                    v0 = plsc.load_gather(table_v, [lo])
                    v1 = plsc.load_gather(table_v, [lo + 16])
                    dst = (row0 + r) * DIM
                    rows[pl.ds(dst, 16)] = v0
                    rows[pl.ds(dst + 16, 16)] = v1
                return carry

            lax.fori_loop(0, n_groups, grp_body, 0)

        def start_wb(chunk, rows, sem):
            pltpu.async_copy(
                rows,
                out_hbm.at[pl.ds((rbase + chunk * CHUNK) * DIM, CHUNK * DIM)],
                sem)

        def wait_wb(rows, sem):
            pltpu.make_async_copy(
                rows, out_hbm.at[pl.ds(0, CHUNK * DIM)], sem).wait()

        def body(g, carry):
            e = 2 * g

            @pl.when(g > 0)
            def _():
                wait_wb(rows0, wb0)

            compute_chunk(e, rows0)

            @pl.when(g > 0)
            def _():
                wait_wb(rows1, wb1)

            start_wb(e, rows0, wb0)
            compute_chunk(e + 1, rows1)
            start_wb(e + 1, rows1, wb1)
            return carry

        lax.fori_loop(0, n_pairs, body, 0)
        wait_wb(rows0, wb0)
        wait_wb(rows1, wb1)

    return emb


def kernel(tensor, table):
    batch, hist = tensor.shape
    b_total = batch * hist
    idx_flat = tensor.reshape(b_total)
    table_out = _build(b_total)(idx_flat, table.reshape(-1))
    return table_out.reshape(batch, hist, DIM)
